# Initial kernel scaffold; baseline (speedup 1.0000x reference)
import jax
import jax.numpy as jnp
from jax.experimental import pallas as pl


def kernel(relative_position_bias_table, relative_position_index):
    raise NotImplementedError("write your pallas kernel here")


# speedup vs baseline: 3.7232x; 3.7232x over previous
import functools

import jax
import jax.numpy as jnp
from jax import lax
from jax.experimental import pallas as pl
from jax.experimental.pallas import tpu as pltpu
from jax.experimental.pallas import tpu_sc as plsc

WH = 24
WW = 24
AREA_P1 = WH * WW + 1
N = AREA_P1 * AREA_P1
NUM_HEADS = 16
TABLE_ROWS = (2 * WH - 1) * (2 * WW - 1) + 3

_INFO = plsc.get_sparse_core_info()
NC = _INFO.num_cores
NS = _INFO.num_subcores
NW = NC * NS
LANES = _INFO.num_lanes

CHUNK = 10416
N_PAD = NW * CHUNK
BLK = 1488
NBLK = CHUNK // BLK
VPB = BLK // LANES


def _sc_bias_gather(table, idx_pad):
    mesh = plsc.VectorSubcoreMesh(core_axis_name="c", subcore_axis_name="s")

    @functools.partial(
        pl.kernel,
        mesh=mesh,
        out_type=jax.ShapeDtypeStruct((NUM_HEADS * N_PAD,), jnp.float32),
        compiler_params=pltpu.CompilerParams(
            needs_layout_passes=False, use_tc_tiling_on_sc=False
        ),
        scratch_types=[
            pltpu.VMEM((TABLE_ROWS * NUM_HEADS,), jnp.float32),
            pltpu.VMEM((CHUNK,), jnp.int32),
            pltpu.VMEM((2, NUM_HEADS, BLK), jnp.float32),
            pltpu.SemaphoreType.DMA,
        ],
    )
    def k(table_hbm, idx_hbm, out_hbm, table_v, idx_v, obuf, sem):
        wid = lax.axis_index("s") * NC + lax.axis_index("c")
        base = wid * CHUNK
        pltpu.sync_copy(table_hbm, table_v)
        pltpu.sync_copy(idx_hbm.at[pl.ds(base, CHUNK)], idx_v)

        pending = [None, None]

        for blk in range(NBLK):
            p = blk % 2
            if pending[p] is not None:
                for d in pending[p]:
                    d.wait()
                pending[p] = None

            off = blk * BLK

            def body(v, _, p=p, off=off):
                idxv = idx_v[pl.ds(off + v * LANES, LANES)] * NUM_HEADS
                for h in range(NUM_HEADS):
                    g = plsc.load_gather(
                        table_v, [idxv + jnp.full((LANES,), h, jnp.int32)]
                    )
                    obuf[p, h, pl.ds(v * LANES, LANES)] = g
                return _

            lax.fori_loop(0, VPB, body, None)

            copies = []
            for h in range(NUM_HEADS):
                copies.append(
                    pltpu.async_copy(
                        obuf.at[p, h],
                        out_hbm.at[pl.ds(h * N_PAD + base + off, BLK)],
                        sem,
                    )
                )
            pending[p] = copies

        for p in range(2):
            if pending[p] is not None:
                for d in pending[p]:
                    d.wait()

    return k(table, idx_pad)


def kernel(relative_position_bias_table, relative_position_index):
    table = relative_position_bias_table.astype(jnp.float32).reshape(-1)
    idx = relative_position_index.reshape(-1).astype(jnp.int32)
    idx_pad = jnp.concatenate(
        [idx, jnp.zeros((N_PAD - N,), jnp.int32)]
    )
    out = _sc_bias_gather(table, idx_pad).reshape(NUM_HEADS, N_PAD)
    return out[:, :N].reshape(NUM_HEADS, AREA_P1, AREA_P1)


# speedup vs baseline: 4.8100x; 1.2919x over previous
import functools

import jax
import jax.numpy as jnp
from jax import lax
from jax.experimental import pallas as pl
from jax.experimental.pallas import tpu as pltpu
from jax.experimental.pallas import tpu_sc as plsc

WH = 24
WW = 24
AREA_P1 = WH * WW + 1
N = AREA_P1 * AREA_P1
NUM_HEADS = 16
TABLE_ROWS = (2 * WH - 1) * (2 * WW - 1) + 3

_INFO = plsc.get_sparse_core_info()
NC = _INFO.num_cores
NS = _INFO.num_subcores
NW = NC * NS
LANES = _INFO.num_lanes

CHUNK = 10416
N_PAD = NW * CHUNK
BLK = 1488
NBLK = CHUNK // BLK
VPB = BLK // LANES


def _sc_bias_gather(table, idx_pad):
    mesh = plsc.VectorSubcoreMesh(core_axis_name="c", subcore_axis_name="s")

    @functools.partial(
        pl.kernel,
        mesh=mesh,
        out_type=jax.ShapeDtypeStruct((NUM_HEADS * N_PAD,), jnp.float32),
        compiler_params=pltpu.CompilerParams(
            needs_layout_passes=False, use_tc_tiling_on_sc=False
        ),
        scratch_types=[
            pltpu.VMEM((TABLE_ROWS * NUM_HEADS,), jnp.float32),
            pltpu.VMEM((CHUNK,), jnp.int32),
            pltpu.VMEM((2, NUM_HEADS, BLK), jnp.float32),
            pltpu.SemaphoreType.DMA,
        ],
    )
    def k(table_hbm, idx_hbm, out_hbm, table_v, idx_v, obuf, sem):
        wid = lax.axis_index("s") * NC + lax.axis_index("c")
        base = wid * CHUNK
        pltpu.sync_copy(table_hbm, table_v)
        pltpu.sync_copy(idx_hbm.at[pl.ds(base, CHUNK)], idx_v)

        pending = [None, None]

        for blk in range(NBLK):
            p = blk % 2
            if pending[p] is not None:
                for d in pending[p]:
                    d.wait()
                pending[p] = None

            off = blk * BLK

            @plsc.parallel_loop(0, VPB, 1, unroll=4)
            def _loop(v, p=p, off=off):
                idxv = idx_v[pl.ds(off + v * LANES, LANES)] * NUM_HEADS
                for h in range(NUM_HEADS):
                    g = plsc.load_gather(
                        table_v, [idxv + jnp.full((LANES,), h, jnp.int32)]
                    )
                    obuf[p, h, pl.ds(v * LANES, LANES)] = g

            copies = []
            for h in range(NUM_HEADS):
                copies.append(
                    pltpu.async_copy(
                        obuf.at[p, h],
                        out_hbm.at[pl.ds(h * N_PAD + base + off, BLK)],
                        sem,
                    )
                )
            pending[p] = copies

        for p in range(2):
            if pending[p] is not None:
                for d in pending[p]:
                    d.wait()

    return k(table, idx_pad)


def kernel(relative_position_bias_table, relative_position_index):
    table = relative_position_bias_table.astype(jnp.float32).reshape(-1)
    idx = relative_position_index.reshape(-1).astype(jnp.int32)
    idx_pad = jnp.concatenate(
        [idx, jnp.zeros((N_PAD - N,), jnp.int32)]
    )
    out = _sc_bias_gather(table, idx_pad).reshape(NUM_HEADS, N_PAD)
    return out[:, :N].reshape(NUM_HEADS, AREA_P1, AREA_P1)
